# baseline (speedup 1.0000x reference)
import functools

import jax
import jax.numpy as jnp
from jax import lax
from jax.experimental import pallas as pl
from jax.experimental.pallas import tpu as pltpu
from jax.experimental.pallas import tpu_sc as plsc

N1 = 1000000
DIM = 16
B = 98304

NUM_CORES = 2
NUM_SUBCORES = 16
NUM_WORKERS = NUM_CORES * NUM_SUBCORES
B_PER_W = B // NUM_WORKERS
PACK = 128 // DIM
LINES_PER_W = B_PER_W // PACK
D_LINES = B // PACK

_UNROLL = 16


def _sc_gather_pair(nodes_embed, context_nodes_embed, source_node, target_node):
    mesh = plsc.VectorSubcoreMesh(core_axis_name="c", subcore_axis_name="s")

    @functools.partial(
        pl.kernel,
        mesh=mesh,
        out_type=[
            jax.ShapeDtypeStruct((D_LINES, 128), jnp.float32),
            jax.ShapeDtypeStruct((D_LINES, 128), jnp.float32),
        ],
        scratch_types=[
            pltpu.VMEM((B_PER_W,), jnp.int32),
            pltpu.VMEM((B_PER_W,), jnp.int32),
            pltpu.VMEM((LINES_PER_W, 128), jnp.float32),
            pltpu.VMEM((LINES_PER_W, 128), jnp.float32),
            pltpu.SemaphoreType.DMA,
            pltpu.SemaphoreType.DMA,
        ],
    )
    def gather_kernel(src_tab, tgt_tab, src_idx, tgt_idx, out_s, out_t,
                      idx_s_v, idx_t_v, rows_s_v, rows_t_v, sem_s, sem_t):
        wid = lax.axis_index("s") * NUM_CORES + lax.axis_index("c")
        base = wid * B_PER_W
        pltpu.sync_copy(src_idx.at[pl.ds(base, B_PER_W)], idx_s_v)
        pltpu.sync_copy(tgt_idx.at[pl.ds(base, B_PER_W)], idx_t_v)

        @pl.loop(0, B_PER_W, step=_UNROLL)
        def _(g):
            line = g // PACK
            ns_vec = idx_s_v[pl.ds(g, _UNROLL)]
            nt_vec = idx_t_v[pl.ds(g, _UNROLL)]
            for j in range(_UNROLL):
                ns = ns_vec[j]
                nt = nt_vec[j]
                ln = line + j // PACK
                dst = pl.ds((j % PACK) * DIM, DIM)
                pltpu.async_copy(src_tab.at[ns], rows_s_v.at[ln, dst], sem_s)
                pltpu.async_copy(tgt_tab.at[nt], rows_t_v.at[ln, dst], sem_t)

        out_slice_s = out_s.at[pl.ds(wid * LINES_PER_W, LINES_PER_W)]
        out_slice_t = out_t.at[pl.ds(wid * LINES_PER_W, LINES_PER_W)]
        pltpu.make_async_copy(out_slice_s, rows_s_v, sem_s).wait()
        pltpu.make_async_copy(out_slice_t, rows_t_v, sem_t).wait()
        pltpu.sync_copy(rows_s_v, out_slice_s)
        pltpu.sync_copy(rows_t_v, out_slice_t)

    return gather_kernel(nodes_embed, context_nodes_embed, source_node,
                         target_node)


_TC_LINES = 2048
_TC_STEPS = D_LINES // _TC_LINES


def _tc_loss_body(s_ref, t_ref, lab_ref, out_ref):
    i = pl.program_id(0)

    @pl.when(i == 0)
    def _():
        out_ref[...] = jnp.zeros_like(out_ref)

    prod = s_ref[...] * t_ref[...]
    for sh in (1, 2, 4, 8):
        prod = prod + jnp.roll(prod, -sh, axis=1)
    z = lab_ref[...] * prod
    loss = jax.nn.log_sigmoid(z)
    lane = lax.broadcasted_iota(jnp.int32, loss.shape, 1)
    picked = jnp.where(lane % DIM == 0, loss, 0.0)
    out_ref[...] += jnp.sum(picked).reshape(1, 1)


def _tc_loss_sum(s_d, t_d, lab_rep):
    return pl.pallas_call(
        _tc_loss_body,
        grid=(_TC_STEPS,),
        in_specs=[
            pl.BlockSpec((_TC_LINES, 128), lambda i: (i, 0)),
            pl.BlockSpec((_TC_LINES, 128), lambda i: (i, 0)),
            pl.BlockSpec((_TC_LINES, 128), lambda i: (i, 0)),
        ],
        out_specs=pl.BlockSpec((1, 1), lambda i: (0, 0)),
        out_shape=jax.ShapeDtypeStruct((1, 1), jnp.float32),
    )(s_d, t_d, lab_rep)


def kernel(source_node, target_node, label, nodes_embed, context_nodes_embed):
    s_d, t_d = _sc_gather_pair(nodes_embed, context_nodes_embed,
                               source_node, target_node)
    lab_rep = jnp.repeat(label, DIM).reshape(D_LINES, 128)
    total = _tc_loss_sum(s_d, t_d, lab_rep)
    return -total[0, 0] / jnp.float32(B)
